# baseline (speedup 1.0000x reference)
import jax
import jax.numpy as jnp
from jax import lax
from jax.experimental import pallas as pl
from jax.experimental.pallas import tpu as pltpu
from jax.experimental.pallas import tpu_sc as plsc

N = 100000
D = 128
S = 256

NC = 2
NS = 16
NW = NC * NS
CHUNK = 125
NCHUNK = 7
RPW = CHUNK * NCHUNK
K = NW * RPW
M = N - K
UNROLL = 5
BB = 912

B = 2000
MB = M // B
PAD2 = 102000


def _sc_body(x_hbm, batch_hbm, w_hbm, b_hbm, pacc_hbm, pden_hbm,
             xb0, xb1, wv, bv, bb, acc, den, sem0, sem1):
    cid = lax.axis_index("c")
    sid = lax.axis_index("s")
    wid = sid * NC + cid
    row0 = M + wid * RPW
    al = (row0 // 8) * 8
    extra = row0 - al

    pltpu.sync_copy(batch_hbm.at[pl.ds(al, BB)], bb)
    pltpu.sync_copy(w_hbm, wv)
    pltpu.sync_copy(b_hbm, bv)

    zero16 = jnp.zeros((16,), jnp.float32)

    def zacc(i, carry):
        acc[i // 8, pl.ds((i % 8) * 16, 16)] = zero16
        return carry

    lax.fori_loop(0, S * (D // 16), zacc, 0)

    def zden(i, carry):
        den[i, pl.ds(0, 16)] = zero16
        return carry

    lax.fori_loop(0, S, zden, 0)

    wvecs = [wv[pl.ds(16 * j, 16)] for j in range(D // 16)]
    bvec = bv[...]
    lane = lax.iota(jnp.int32, 16)

    def _slice(c):
        return x_hbm.at[pl.ds((row0 + c * CHUNK) * D, CHUNK * D)]

    def start(c, buf, sem):
        pltpu.async_copy(_slice(c), buf, sem)

    def wait(c, buf, sem):
        pltpu.make_async_copy(_slice(c), buf, sem).wait()

    def group(buf, c, q):
        r0 = q * UNROLL
        segs = [bb[pl.ds(extra + c * CHUNK + r0 + u, 16)][0] for u in range(UNROLL)]
        xvs = [[buf[pl.ds((r0 + u) * D + 16 * j, 16)] for j in range(D // 16)]
               for u in range(UNROLL)]
        parts = []
        for u in range(UNROLL):
            prods = [xvs[u][j] * wvecs[j] for j in range(D // 16)]
            while len(prods) > 1:
                prods = [prods[i] + prods[i + 1] for i in range(0, len(prods), 2)]
            parts.append(prods[0])
        for k in (8, 4, 2, 1):
            parts = [p + jnp.take(p, lane ^ k, axis=0) for p in parts]
        evs = [jnp.exp(bvec + p) for p in parts]
        for u in range(UNROLL):
            for j in range(D // 16):
                plsc.addupdate(acc.at[segs[u], pl.ds(16 * j, 16)], evs[u] * xvs[u][j])
            plsc.addupdate(den.at[segs[u], pl.ds(0, 16)], evs[u])

    def process(buf, c):
        def rb(q, carry):
            group(buf, c, q)
            return carry

        lax.fori_loop(0, CHUNK // UNROLL, rb, 0)

    start(0, xb0, sem0)

    def pair_body(i, carry):
        c0 = 2 * i
        start(c0 + 1, xb1, sem1)
        wait(c0, xb0, sem0)
        process(xb0, c0)
        start(c0 + 2, xb0, sem0)
        wait(c0 + 1, xb1, sem1)
        process(xb1, c0 + 1)
        return carry

    lax.fori_loop(0, (NCHUNK - 1) // 2, pair_body, 0)
    wait(NCHUNK - 1, xb0, sem0)
    process(xb0, NCHUNK - 1)

    pltpu.sync_copy(acc, pacc_hbm.at[pl.ds(wid * S, S)])
    pltpu.sync_copy(den, pden_hbm.at[pl.ds(wid * S, S)])


def _sc_pool(xflat, batch_pad, wflat, b16):
    mesh = plsc.VectorSubcoreMesh(core_axis_name="c", subcore_axis_name="s")
    f = pl.kernel(
        _sc_body,
        out_type=(
            jax.ShapeDtypeStruct((NW * S, D), jnp.float32),
            jax.ShapeDtypeStruct((NW * S, 16), jnp.float32),
        ),
        mesh=mesh,
        scratch_types=[
            pltpu.VMEM((CHUNK * D,), jnp.float32),
            pltpu.VMEM((CHUNK * D,), jnp.float32),
            pltpu.VMEM((D,), jnp.float32),
            pltpu.VMEM((16,), jnp.float32),
            pltpu.VMEM((BB,), jnp.int32),
            pltpu.VMEM((S, D), jnp.float32),
            pltpu.VMEM((S, 16), jnp.float32),
            pltpu.SemaphoreType.DMA,
            pltpu.SemaphoreType.DMA,
        ],
    )
    return f(xflat, batch_pad, wflat, b16)


def _tc_body(batch_ref, x_ref, w8_ref, b_ref, num_out, den_out, num_ref, den_ref):
    i = pl.program_id(0)

    @pl.when(i == 0)
    def _():
        num_ref[...] = jnp.zeros_like(num_ref)
        den_ref[...] = jnp.zeros_like(den_ref)

    x = x_ref[...]
    g8 = jax.lax.dot(x, w8_ref[...], preferred_element_type=jnp.float32)
    e = jnp.exp(g8[:, 0:1] + b_ref[0, 0])
    bv = batch_ref[0]
    ids = jax.lax.broadcasted_iota(jnp.int32, (S, B), 0)
    oh = (ids == bv).astype(jnp.bfloat16)
    xe = (x * e).astype(jnp.bfloat16)
    num_ref[...] += jax.lax.dot(oh, xe, preferred_element_type=jnp.float32)
    e8 = jnp.broadcast_to(e, (B, 8)).astype(jnp.bfloat16)
    den_ref[...] += jax.lax.dot(oh, e8, preferred_element_type=jnp.float32)

    @pl.when(i == MB - 1)
    def _():
        num_out[...] = num_ref[...]
        den_out[...] = den_ref[...]


def _tc_pool(x, batchT, W8, b2):
    return pl.pallas_call(
        _tc_body,
        grid=(MB,),
        in_specs=[
            pl.BlockSpec((1, 1, B), lambda i: (i, 0, 0)),
            pl.BlockSpec((B, D), lambda i: (i, 0)),
            pl.BlockSpec((D, 8), lambda i: (0, 0)),
            pl.BlockSpec((1, 1), lambda i: (0, 0)),
        ],
        out_specs=[
            pl.BlockSpec((S, D), lambda i: (0, 0)),
            pl.BlockSpec((S, 8), lambda i: (0, 0)),
        ],
        out_shape=[
            jax.ShapeDtypeStruct((S, D), jnp.float32),
            jax.ShapeDtypeStruct((S, 8), jnp.float32),
        ],
        scratch_shapes=[
            pltpu.VMEM((S, D), jnp.float32),
            pltpu.VMEM((S, 8), jnp.float32),
        ],
        compiler_params=pltpu.CompilerParams(
            dimension_semantics=("arbitrary",),
        ),
    )(batchT, x, W8, b2)


def _ep_body(pacc_ref, pden_ref, num_ref, den8_ref, out_ref):
    s = num_ref[...] + jnp.sum(pacc_ref[...], axis=0)
    dn = den8_ref[..., 0:1] + jnp.sum(pden_ref[..., 0:1], axis=0)
    out_ref[...] = s / jnp.maximum(dn, 1e-30)


def kernel(x, batch, W, b):
    batch_pad = jnp.pad(batch.astype(jnp.int32), (0, PAD2 - N))
    batchT = batch_pad.reshape(PAD2 // B, 1, B)
    wflat = W.reshape(D).astype(jnp.float32)
    b16 = jnp.broadcast_to(b.astype(jnp.float32), (16,))
    W8 = jnp.pad(W.reshape(D, 1).astype(jnp.float32), ((0, 0), (0, 7)))
    b2 = b.reshape(1, 1).astype(jnp.float32)
    pacc, pden = _sc_pool(x.reshape(N * D), batch_pad, wflat, b16)
    num_tc, den_tc = _tc_pool(x, batchT, W8, b2)
    out = pl.pallas_call(
        _ep_body,
        out_shape=jax.ShapeDtypeStruct((S, D), jnp.float32),
    )(pacc.reshape(NW, S, D), pden.reshape(NW, S, 16), num_tc, den_tc)
    return out


# speedup vs baseline: 1.3234x; 1.3234x over previous
import jax
import jax.numpy as jnp
from jax import lax
from jax.experimental import pallas as pl
from jax.experimental.pallas import tpu as pltpu
from jax.experimental.pallas import tpu_sc as plsc

N = 100000
D = 128
S = 256

NC = 2
NS = 16
NW = NC * NS
CHUNK = 125
NCHUNK = 9
RPW = CHUNK * NCHUNK
K = NW * RPW
UNROLL = 5
BB = 1152

B = 4000
MB = (N - K) // B
OFF = K // B


def _sc_body(x_hbm, batch_hbm, w_hbm, b_hbm, pacc_hbm, pden_hbm,
             xb0, xb1, wv, bv, bb, acc, den, sem0, sem1):
    cid = lax.axis_index("c")
    sid = lax.axis_index("s")
    wid = sid * NC + cid
    row0 = wid * RPW
    al = (row0 // 8) * 8
    extra = row0 - al

    pltpu.sync_copy(batch_hbm.at[pl.ds(al, BB)], bb)
    pltpu.sync_copy(w_hbm, wv)
    pltpu.sync_copy(b_hbm, bv)

    zero16 = jnp.zeros((16,), jnp.float32)

    def zacc(i, carry):
        acc[i // 8, pl.ds((i % 8) * 16, 16)] = zero16
        return carry

    lax.fori_loop(0, S * (D // 16), zacc, 0)

    def zden(i, carry):
        den[i, pl.ds(0, 16)] = zero16
        return carry

    lax.fori_loop(0, S, zden, 0)

    wvecs = [wv[pl.ds(16 * j, 16)] for j in range(D // 16)]
    bvec = bv[...]
    lane = lax.iota(jnp.int32, 16)

    def _slice(c):
        return x_hbm.at[pl.ds((row0 + c * CHUNK) * D, CHUNK * D)]

    def start(c, buf, sem):
        pltpu.async_copy(_slice(c), buf, sem)

    def wait(c, buf, sem):
        pltpu.make_async_copy(_slice(c), buf, sem).wait()

    def group(buf, c, q):
        r0 = q * UNROLL
        segs = [bb[pl.ds(extra + c * CHUNK + r0 + u, 16)][0] for u in range(UNROLL)]
        xvs = [[buf[pl.ds((r0 + u) * D + 16 * j, 16)] for j in range(D // 16)]
               for u in range(UNROLL)]
        parts = []
        for u in range(UNROLL):
            prods = [xvs[u][j] * wvecs[j] for j in range(D // 16)]
            while len(prods) > 1:
                prods = [prods[i] + prods[i + 1] for i in range(0, len(prods), 2)]
            parts.append(prods[0])
        for k in (8, 4, 2, 1):
            parts = [p + jnp.take(p, lane ^ k, axis=0) for p in parts]
        evs = [jnp.exp(bvec + p) for p in parts]
        for u in range(UNROLL):
            for j in range(D // 16):
                plsc.addupdate(acc.at[segs[u], pl.ds(16 * j, 16)], evs[u] * xvs[u][j])
            plsc.addupdate(den.at[segs[u], pl.ds(0, 16)], evs[u])

    def process(buf, c):
        def rb(q, carry):
            group(buf, c, q)
            return carry

        lax.fori_loop(0, CHUNK // UNROLL, rb, 0)

    start(0, xb0, sem0)

    def pair_body(i, carry):
        c0 = 2 * i
        start(c0 + 1, xb1, sem1)
        wait(c0, xb0, sem0)
        process(xb0, c0)
        start(c0 + 2, xb0, sem0)
        wait(c0 + 1, xb1, sem1)
        process(xb1, c0 + 1)
        return carry

    lax.fori_loop(0, (NCHUNK - 1) // 2, pair_body, 0)
    wait(NCHUNK - 1, xb0, sem0)
    process(xb0, NCHUNK - 1)

    pltpu.sync_copy(acc, pacc_hbm.at[pl.ds(wid * S, S)])
    pltpu.sync_copy(den, pden_hbm.at[pl.ds(wid * S, S)])


def _sc_pool(xflat, batch, wflat, b16):
    mesh = plsc.VectorSubcoreMesh(core_axis_name="c", subcore_axis_name="s")
    f = pl.kernel(
        _sc_body,
        out_type=(
            jax.ShapeDtypeStruct((NW * S, D), jnp.float32),
            jax.ShapeDtypeStruct((NW * S, 16), jnp.float32),
        ),
        mesh=mesh,
        scratch_types=[
            pltpu.VMEM((CHUNK * D,), jnp.float32),
            pltpu.VMEM((CHUNK * D,), jnp.float32),
            pltpu.VMEM((D,), jnp.float32),
            pltpu.VMEM((16,), jnp.float32),
            pltpu.VMEM((BB,), jnp.int32),
            pltpu.VMEM((S, D), jnp.float32),
            pltpu.VMEM((S, 16), jnp.float32),
            pltpu.SemaphoreType.DMA,
            pltpu.SemaphoreType.DMA,
        ],
    )
    return f(xflat, batch, wflat, b16)


def _tc_body(batch_ref, x_ref, w8_ref, b_ref, num_out, den_out, acc_ref):
    i = pl.program_id(0)

    @pl.when(i == 0)
    def _():
        acc_ref[...] = jnp.zeros_like(acc_ref)

    x = x_ref[...]
    g8 = jax.lax.dot(x, w8_ref[...], preferred_element_type=jnp.float32)
    e = jnp.exp(g8[:, 0:1] + b_ref[0, 0])
    bv = batch_ref[0]
    ids = jax.lax.broadcasted_iota(jnp.int32, (S, B), 0)
    oh = (ids == bv).astype(jnp.bfloat16)
    xe = (x * e).astype(jnp.bfloat16)
    e8 = jnp.broadcast_to(e, (B, 8)).astype(jnp.bfloat16)
    xep = jnp.concatenate([xe, e8], axis=1)
    acc_ref[...] += jax.lax.dot(oh, xep, preferred_element_type=jnp.float32)

    @pl.when(i == MB - 1)
    def _():
        num_out[...] = acc_ref[:, :D]
        den_out[...] = acc_ref[:, D:]


def _tc_pool(x, batchT, W8, b2):
    return pl.pallas_call(
        _tc_body,
        grid=(MB,),
        in_specs=[
            pl.BlockSpec((1, 1, B), lambda i: (i + OFF, 0, 0)),
            pl.BlockSpec((B, D), lambda i: (i + OFF, 0)),
            pl.BlockSpec((D, 8), lambda i: (0, 0)),
            pl.BlockSpec((1, 1), lambda i: (0, 0)),
        ],
        out_specs=[
            pl.BlockSpec((S, D), lambda i: (0, 0)),
            pl.BlockSpec((S, 8), lambda i: (0, 0)),
        ],
        out_shape=[
            jax.ShapeDtypeStruct((S, D), jnp.float32),
            jax.ShapeDtypeStruct((S, 8), jnp.float32),
        ],
        scratch_shapes=[
            pltpu.VMEM((S, D + 8), jnp.float32),
        ],
        compiler_params=pltpu.CompilerParams(
            dimension_semantics=("arbitrary",),
        ),
    )(batchT, x, W8, b2)


def _ep_body(pacc_ref, pden_ref, num_ref, den8_ref, out_ref):
    s = num_ref[...] + jnp.sum(pacc_ref[...], axis=0)
    dn = den8_ref[..., 0:1] + jnp.sum(pden_ref[..., 0:1], axis=0)
    out_ref[...] = s / jnp.maximum(dn, 1e-30)


def kernel(x, batch, W, b):
    batch32 = batch.astype(jnp.int32)
    batchT = batch32.reshape(N // B, 1, B)
    wflat = W.reshape(D).astype(jnp.float32)
    b16 = jnp.broadcast_to(b.astype(jnp.float32), (16,))
    W8 = jnp.pad(W.reshape(D, 1).astype(jnp.float32), ((0, 0), (0, 7)))
    b2 = b.reshape(1, 1).astype(jnp.float32)
    pacc, pden = _sc_pool(x.reshape(N * D), batch32, wflat, b16)
    num_tc, den_tc = _tc_pool(x, batchT, W8, b2)
    out = pl.pallas_call(
        _ep_body,
        out_shape=jax.ShapeDtypeStruct((S, D), jnp.float32),
    )(pacc.reshape(NW, S, D), pden.reshape(NW, S, 16), num_tc, den_tc)
    return out
